# Initial kernel scaffold; baseline (speedup 1.0000x reference)
import functools
import numpy as np
import jax
import jax.numpy as jnp
from jax import lax
from jax.experimental import pallas as pl
from jax.experimental.pallas import tpu as pltpu
from jax.experimental.pallas import tpu_sc as plsc

_N, _LQ, _C, _L, _M, _P = 4, 2048, 256, 4, 8, 4
_D = _C // _M
_LENS = (2048, 1024, 512, 256)
_LV = sum(_LENS)
_STARTS = (0, 2048, 3072, 3584)
_LP = _L * _P
_QC = 64
_QB = 512

_T_LANE = np.tile(np.repeat(np.array(_LENS, np.float32), _P), _M).reshape(1, 128)
_S_LANE = np.tile(np.repeat(np.array(_STARTS, np.int32), _P), _M).reshape(1, 128)
_SEL = np.zeros((_L, 128), np.float32)
for _lane in range(128):
    _SEL[(_lane % _LP) // _P, _lane] = 1.0


def _mm_bias_kernel(x_ref, w_ref, b_ref, o_ref):
    o_ref[...] = (
        jnp.dot(x_ref[...], w_ref[...], preferred_element_type=jnp.float32)
        + b_ref[...]
    )


def _mm_bias(x, wT, b, rb):
    rows, k = x.shape
    o = wT.shape[1]
    return pl.pallas_call(
        _mm_bias_kernel,
        grid=(rows // rb,),
        in_specs=[
            pl.BlockSpec((rb, k), lambda i: (i, 0)),
            pl.BlockSpec((k, o), lambda i: (0, 0)),
            pl.BlockSpec((1, o), lambda i: (0, 0)),
        ],
        out_specs=pl.BlockSpec((rb, o), lambda i: (i, 0)),
        out_shape=jax.ShapeDtypeStruct((rows, o), jnp.float32),
    )(x, wT, b.reshape(1, o))


def _prep_kernel(q_ref, rp_ref, woffT_ref, boff_ref, wattnT_ref, battn_ref,
                 loc_ref, aw_ref, idx0_ref, idx1_ref, ww0_ref, ww1_ref):
    q = q_ref[...]
    qb = q.shape[0]
    off = jnp.dot(q, woffT_ref[...], preferred_element_type=jnp.float32) + boff_ref[...]
    a = jnp.dot(q, wattnT_ref[...], preferred_element_type=jnp.float32) + battn_ref[...]
    a3 = a.reshape(qb, _M, _LP)
    amax = jnp.max(a3, axis=-1, keepdims=True)
    e = jnp.exp(a3 - amax)
    aw = (e / jnp.sum(e, axis=-1, keepdims=True)).reshape(qb, 128)
    aw_ref[...] = aw

    t_lane = jnp.asarray(_T_LANE)
    ref_lane = jnp.dot(rp_ref[...], jnp.asarray(_SEL),
                       preferred_element_type=jnp.float32)
    loc = ref_lane + off / t_lane
    loc_ref[...] = loc

    xg = 2.0 * loc - 1.0
    x = ((xg + 1.0) * t_lane - 1.0) * 0.5
    x0 = jnp.floor(x)
    w1 = x - x0
    w0 = 1.0 - w1
    x0i = x0.astype(jnp.int32)
    x1i = x0i + 1
    tl_i = t_lane.astype(jnp.int32)
    s_lane = jnp.asarray(_S_LANE)
    v0 = ((x0i >= 0) & (x0i < tl_i)).astype(jnp.float32)
    v1 = ((x1i >= 0) & (x1i < tl_i)).astype(jnp.float32)
    idx0_ref[...] = jnp.clip(x0i, 0, tl_i - 1) + s_lane
    idx1_ref[...] = jnp.clip(x1i, 0, tl_i - 1) + s_lane
    ww0_ref[...] = aw * w0 * v0
    ww1_ref[...] = aw * w1 * v1


def _prep(query2d, refp2d, WoffT, boff, WattnT, battn):
    rows = query2d.shape[0]
    f32 = jnp.float32
    out_shapes = [
        jax.ShapeDtypeStruct((rows, 128), f32),
        jax.ShapeDtypeStruct((rows, 128), f32),
        jax.ShapeDtypeStruct((rows, 128), jnp.int32),
        jax.ShapeDtypeStruct((rows, 128), jnp.int32),
        jax.ShapeDtypeStruct((rows, 128), f32),
        jax.ShapeDtypeStruct((rows, 128), f32),
    ]
    vec_spec = pl.BlockSpec((_QB, 128), lambda i: (i, 0))
    return pl.pallas_call(
        _prep_kernel,
        grid=(rows // _QB,),
        in_specs=[
            pl.BlockSpec((_QB, _C), lambda i: (i, 0)),
            pl.BlockSpec((_QB, _L), lambda i: (i, 0)),
            pl.BlockSpec((_C, 128), lambda i: (0, 0)),
            pl.BlockSpec((1, 128), lambda i: (0, 0)),
            pl.BlockSpec((_C, 128), lambda i: (0, 0)),
            pl.BlockSpec((1, 128), lambda i: (0, 0)),
        ],
        out_specs=[vec_spec] * 6,
        out_shape=out_shapes,
    )(query2d, refp2d, WoffT, boff.reshape(1, 128), WattnT, battn.reshape(1, 128))


def _sample_sc(value, idx0, idx1, ww0, ww1):
    mesh = plsc.VectorSubcoreMesh(core_axis_name="c", subcore_axis_name="s")

    @functools.partial(
        pl.kernel,
        mesh=mesh,
        out_type=jax.ShapeDtypeStruct((_N, _LQ, _C), jnp.float32),
        scratch_types=[
            pltpu.VMEM((_LV, _D), jnp.float32),
            pltpu.VMEM((_QC, _LP), jnp.int32),
            pltpu.VMEM((_QC, _LP), jnp.int32),
            pltpu.VMEM((_QC, _LP), jnp.float32),
            pltpu.VMEM((_QC, _LP), jnp.float32),
            pltpu.VMEM((_QC, _D), jnp.float32),
        ],
    )
    def k(value_hbm, idx0_hbm, idx1_hbm, ww0_hbm, ww1_hbm, out_hbm,
          val_v, i0_v, i1_v, w0_v, w1_v, out_v):
        cid = lax.axis_index("c")
        sid = lax.axis_index("s")
        wid = sid * 2 + cid
        n = wid // _M
        m = wid % _M
        pltpu.sync_copy(value_hbm.at[n, :, pl.ds(m * _D, _D)], val_v)

        def chunk_body(ch, carry):
            q0 = ch * _QC
            pltpu.sync_copy(idx0_hbm.at[n, pl.ds(q0, _QC), pl.ds(m * _LP, _LP)], i0_v)
            pltpu.sync_copy(idx1_hbm.at[n, pl.ds(q0, _QC), pl.ds(m * _LP, _LP)], i1_v)
            pltpu.sync_copy(ww0_hbm.at[n, pl.ds(q0, _QC), pl.ds(m * _LP, _LP)], w0_v)
            pltpu.sync_copy(ww1_hbm.at[n, pl.ds(q0, _QC), pl.ds(m * _LP, _LP)], w1_v)

            def q_body(qq, c2):
                acc0 = jnp.zeros((16,), jnp.float32)
                acc1 = jnp.zeros((16,), jnp.float32)
                for t in range(_LP):
                    r0 = i0_v[qq, t]
                    w0s = w0_v[qq, t]
                    acc0 = acc0 + w0s * val_v[r0, pl.ds(0, 16)]
                    acc1 = acc1 + w0s * val_v[r0, pl.ds(16, 16)]
                    r1 = i1_v[qq, t]
                    w1s = w1_v[qq, t]
                    acc0 = acc0 + w1s * val_v[r1, pl.ds(0, 16)]
                    acc1 = acc1 + w1s * val_v[r1, pl.ds(16, 16)]
                out_v[qq, pl.ds(0, 16)] = acc0
                out_v[qq, pl.ds(16, 16)] = acc1
                return c2

            lax.fori_loop(0, _QC, q_body, 0)
            pltpu.sync_copy(out_v, out_hbm.at[n, pl.ds(q0, _QC), pl.ds(m * _D, _D)])
            return carry

        lax.fori_loop(0, _LQ // _QC, chunk_body, 0)

    return k(value, idx0, idx1, ww0, ww1)


def kernel(query, reference_points, input_flatten, input_temporal_lens,
           input_level_start_index, Wv, bv, Woff, boff, Wattn, battn,
           Wout, bout):
    n, lq, c = query.shape
    value = _mm_bias(input_flatten.reshape(n * _LV, c), Wv.T, bv, 512)
    value = value.reshape(n, _LV, c)

    loc, aw, idx0, idx1, ww0, ww1 = _prep(
        query.reshape(n * lq, c),
        reference_points.reshape(n * lq, _L),
        Woff.T, boff, Wattn.T, battn,
    )

    heads = _sample_sc(
        value,
        idx0.reshape(n, lq, 128), idx1.reshape(n, lq, 128),
        ww0.reshape(n, lq, 128), ww1.reshape(n, lq, 128),
    )

    out = _mm_bias(heads.reshape(n * lq, c), Wout.T, bout, 512)
    out = out.reshape(n, lq, c)

    loc6 = loc.reshape(n, lq, _M, _L, _P, 1)
    sampling_locations = jnp.concatenate(
        [loc6, jnp.full_like(loc6, 0.5)], axis=-1)
    aw_out = aw.reshape(n, lq, _M, _L, _P)
    return out, sampling_locations, aw_out


# speedup vs baseline: 1557.2495x; 1557.2495x over previous
import functools
import numpy as np
import jax
import jax.numpy as jnp
from jax import lax
from jax.experimental import pallas as pl
from jax.experimental.pallas import tpu as pltpu
from jax.experimental.pallas import tpu_sc as plsc

_N, _LQ, _C, _L, _M, _P = 4, 2048, 256, 4, 8, 4
_D = _C // _M
_LENS = (2048, 1024, 512, 256)
_LV = sum(_LENS)
_STARTS = (0, 2048, 3072, 3584)
_LP = _L * _P
_QC = 64
_QB = 512

_T_LANE = np.tile(np.repeat(np.array(_LENS, np.float32), _P), _M).reshape(1, 128)
_S_LANE = np.tile(np.repeat(np.array(_STARTS, np.int32), _P), _M).reshape(1, 128)
_SEL = np.zeros((_L, 128), np.float32)
for _lane in range(128):
    _SEL[(_lane % _LP) // _P, _lane] = 1.0


def _mm_bias_kernel(x_ref, w_ref, b_ref, o_ref):
    o_ref[...] = (
        jnp.dot(x_ref[...], w_ref[...], preferred_element_type=jnp.float32, precision=lax.Precision.HIGHEST)
        + b_ref[...]
    )


def _mm_bias(x, wT, b, rb):
    rows, k = x.shape
    o = wT.shape[1]
    return pl.pallas_call(
        _mm_bias_kernel,
        grid=(rows // rb,),
        in_specs=[
            pl.BlockSpec((rb, k), lambda i: (i, 0)),
            pl.BlockSpec((k, o), lambda i: (0, 0)),
            pl.BlockSpec((1, o), lambda i: (0, 0)),
        ],
        out_specs=pl.BlockSpec((rb, o), lambda i: (i, 0)),
        out_shape=jax.ShapeDtypeStruct((rows, o), jnp.float32),
    )(x, wT, b.reshape(1, o))


def _prep_kernel(q_ref, rp_ref, woffT_ref, boff_ref, wattnT_ref, battn_ref,
                 tlane_ref, slane_ref, sel_ref,
                 loc_ref, aw_ref, idx0_ref, idx1_ref, ww0_ref, ww1_ref):
    q = q_ref[...]
    qb = q.shape[0]
    off = jnp.dot(q, woffT_ref[...], preferred_element_type=jnp.float32, precision=lax.Precision.HIGHEST) + boff_ref[...]
    a = jnp.dot(q, wattnT_ref[...], preferred_element_type=jnp.float32, precision=lax.Precision.HIGHEST) + battn_ref[...]
    a3 = a.reshape(qb, _M, _LP)
    amax = jnp.max(a3, axis=-1, keepdims=True)
    e = jnp.exp(a3 - amax)
    aw = (e / jnp.sum(e, axis=-1, keepdims=True)).reshape(qb, 128)
    aw_ref[...] = aw

    t_lane = tlane_ref[...]
    ref_lane = jnp.dot(rp_ref[...], sel_ref[...],
                       preferred_element_type=jnp.float32, precision=lax.Precision.HIGHEST)
    loc = ref_lane + off / t_lane
    loc_ref[...] = loc

    xg = 2.0 * loc - 1.0
    x = ((xg + 1.0) * t_lane - 1.0) * 0.5
    x0 = jnp.floor(x)
    w1 = x - x0
    w0 = 1.0 - w1
    x0i = x0.astype(jnp.int32)
    x1i = x0i + 1
    tl_i = t_lane.astype(jnp.int32)
    s_lane = slane_ref[...]
    v0 = ((x0i >= 0) & (x0i < tl_i)).astype(jnp.float32)
    v1 = ((x1i >= 0) & (x1i < tl_i)).astype(jnp.float32)
    idx0_ref[...] = jnp.clip(x0i, 0, tl_i - 1) + s_lane
    idx1_ref[...] = jnp.clip(x1i, 0, tl_i - 1) + s_lane
    ww0_ref[...] = aw * w0 * v0
    ww1_ref[...] = aw * w1 * v1


def _prep(query2d, refp2d, WoffT, boff, WattnT, battn):
    rows = query2d.shape[0]
    f32 = jnp.float32
    out_shapes = [
        jax.ShapeDtypeStruct((rows, 128), f32),
        jax.ShapeDtypeStruct((rows, 128), f32),
        jax.ShapeDtypeStruct((rows, 128), jnp.int32),
        jax.ShapeDtypeStruct((rows, 128), jnp.int32),
        jax.ShapeDtypeStruct((rows, 128), f32),
        jax.ShapeDtypeStruct((rows, 128), f32),
    ]
    vec_spec = pl.BlockSpec((_QB, 128), lambda i: (i, 0))
    return pl.pallas_call(
        _prep_kernel,
        grid=(rows // _QB,),
        in_specs=[
            pl.BlockSpec((_QB, _C), lambda i: (i, 0)),
            pl.BlockSpec((_QB, _L), lambda i: (i, 0)),
            pl.BlockSpec((_C, 128), lambda i: (0, 0)),
            pl.BlockSpec((1, 128), lambda i: (0, 0)),
            pl.BlockSpec((_C, 128), lambda i: (0, 0)),
            pl.BlockSpec((1, 128), lambda i: (0, 0)),
            pl.BlockSpec((1, 128), lambda i: (0, 0)),
            pl.BlockSpec((1, 128), lambda i: (0, 0)),
            pl.BlockSpec((_L, 128), lambda i: (0, 0)),
        ],
        out_specs=[vec_spec] * 6,
        out_shape=out_shapes,
    )(query2d, refp2d, WoffT, boff.reshape(1, 128), WattnT, battn.reshape(1, 128),
      jnp.asarray(_T_LANE), jnp.asarray(_S_LANE), jnp.asarray(_SEL))


def _sample_sc(value, idx0, idx1, ww0, ww1):
    mesh = plsc.VectorSubcoreMesh(core_axis_name="c", subcore_axis_name="s",
                                  num_cores=2, num_subcores=16)

    @functools.partial(
        pl.kernel,
        mesh=mesh,
        compiler_params=pltpu.CompilerParams(use_tc_tiling_on_sc=False),
        out_type=jax.ShapeDtypeStruct((_N, _M, _LQ, _D), jnp.float32),
        scratch_types=[
            pltpu.VMEM((_LV, _D), jnp.float32),
            pltpu.VMEM((_QC, _LP), jnp.int32),
            pltpu.VMEM((_QC, _LP), jnp.int32),
            pltpu.VMEM((_QC, _LP), jnp.float32),
            pltpu.VMEM((_QC, _LP), jnp.float32),
            pltpu.VMEM((_QC, _D), jnp.float32),
        ],
    )
    def k(value_hbm, idx0_hbm, idx1_hbm, ww0_hbm, ww1_hbm, out_hbm,
          val_v, i0_v, i1_v, w0_v, w1_v, out_v):
        cid = lax.axis_index("c")
        sid = lax.axis_index("s")
        wid = sid * 2 + cid
        n = wid // _M
        m = wid % _M
        pltpu.sync_copy(value_hbm.at[n, m], val_v)

        def chunk_body(ch, carry):
            q0 = ch * _QC
            pltpu.sync_copy(idx0_hbm.at[n, m, pl.ds(q0, _QC), :], i0_v)
            pltpu.sync_copy(idx1_hbm.at[n, m, pl.ds(q0, _QC), :], i1_v)
            pltpu.sync_copy(ww0_hbm.at[n, m, pl.ds(q0, _QC), :], w0_v)
            pltpu.sync_copy(ww1_hbm.at[n, m, pl.ds(q0, _QC), :], w1_v)

            def q_body(qq, c2):
                acc0 = jnp.zeros((16,), jnp.float32)
                acc1 = jnp.zeros((16,), jnp.float32)
                iv0 = i0_v[qq, pl.ds(0, _LP)]
                iv1 = i1_v[qq, pl.ds(0, _LP)]
                wv0 = w0_v[qq, pl.ds(0, _LP)]
                wv1 = w1_v[qq, pl.ds(0, _LP)]
                for t in range(_LP):
                    r0 = iv0[t]
                    w0s = wv0[t]
                    acc0 = acc0 + w0s * val_v[r0, pl.ds(0, 16)]
                    acc1 = acc1 + w0s * val_v[r0, pl.ds(16, 16)]
                    r1 = iv1[t]
                    w1s = wv1[t]
                    acc0 = acc0 + w1s * val_v[r1, pl.ds(0, 16)]
                    acc1 = acc1 + w1s * val_v[r1, pl.ds(16, 16)]
                out_v[qq, pl.ds(0, 16)] = acc0
                out_v[qq, pl.ds(16, 16)] = acc1
                return c2

            lax.fori_loop(0, _QC, q_body, 0)
            pltpu.sync_copy(out_v, out_hbm.at[n, m, pl.ds(q0, _QC), :])
            return carry

        lax.fori_loop(0, _LQ // _QC, chunk_body, 0)

    return k(value, idx0, idx1, ww0, ww1)


def kernel(query, reference_points, input_flatten, input_temporal_lens,
           input_level_start_index, Wv, bv, Woff, boff, Wattn, battn,
           Wout, bout):
    n, lq, c = query.shape
    value = _mm_bias(input_flatten.reshape(n * _LV, c), Wv.T, bv, 512)
    value = value.reshape(n, _LV, _M, _D).transpose(0, 2, 1, 3)

    loc, aw, idx0, idx1, ww0, ww1 = _prep(
        query.reshape(n * lq, c),
        reference_points.reshape(n * lq, _L),
        Woff.T, boff, Wattn.T, battn,
    )

    def _hm(x):
        return x.reshape(n, lq, _M, _LP).transpose(0, 2, 1, 3)

    heads = _sample_sc(value, _hm(idx0), _hm(idx1), _hm(ww0), _hm(ww1))
    heads = heads.transpose(0, 2, 1, 3)

    out = _mm_bias(heads.reshape(n * lq, c), Wout.T, bout, 512)
    out = out.reshape(n, lq, c)

    loc6 = loc.reshape(n, lq, _M, _L, _P, 1)
    sampling_locations = jnp.concatenate(
        [loc6, jnp.full_like(loc6, 0.5)], axis=-1)
    aw_out = aw.reshape(n, lq, _M, _L, _P)
    return out, sampling_locations, aw_out
